# baseline (speedup 1.0000x reference)
import functools

import jax
import jax.numpy as jnp
from jax import lax
from jax.experimental import pallas as pl
from jax.experimental.pallas import tpu as pltpu
from jax.experimental.pallas import tpu_sc as plsc

FEAT = 256
QS = 16384
BATCH = 1024
INV_T = 1.0 / 0.07
CW = 512
NCB = (QS + 1 + CW - 1) // CW
NMB = QS // CW


def _body(q_ref, mem_ref, ksf_ref, kdf1_ref, kdf2_ref,
          out_ref, lpos_ref, carry_ref):
    c = pl.program_id(0)
    q = q_ref[...]
    m = mem_ref[...]

    prev = jnp.where(c == 0, jnp.zeros((1, FEAT), jnp.float32), carry_ref[...])
    rhs = jnp.concatenate([prev, m[:CW - 1]], axis=0)
    carry_ref[...] = m[CW - 1:CW]

    tile = jax.lax.dot_general(
        q, rhs, (((1,), (1,)), ((), ())),
        preferred_element_type=jnp.float32,
    ) * INV_T
    out_ref[0:BATCH] = tile
    out_ref[BATCH:2 * BATCH] = tile
    out_ref[2 * BATCH:3 * BATCH] = tile

    @pl.when(c == 0)
    def _():
        psf = jnp.sum(q * ksf_ref[...], axis=1, keepdims=True)
        pd1 = jnp.sum(q * kdf1_ref[...], axis=1, keepdims=True)
        pd2 = jnp.sum(q * kdf2_ref[...], axis=1, keepdims=True)
        lpos_ref[...] = psf
        out_ref[0:BATCH, 0:1] = psf * INV_T
        out_ref[BATCH:2 * BATCH, 0:1] = pd1 * INV_T
        out_ref[2 * BATCH:3 * BATCH, 0:1] = pd2 * INV_T

def _queue_update_sc(k_all_sf, k_all_df1, k_all_df2, memory):
    info = plsc.get_sparse_core_info()
    nc = info.num_cores
    rows_per_w = QS // (nc * info.num_subcores)
    mesh = plsc.VectorSubcoreMesh(core_axis_name="c", subcore_axis_name="s")

    rchunk = 128

    @functools.partial(
        pl.kernel, mesh=mesh,
        out_type=jax.ShapeDtypeStruct((QS, FEAT), jnp.float32),
        scratch_types=[pltpu.VMEM((rchunk, FEAT), jnp.float32)],
    )
    def upd(kasf, kadf1, kadf2, mem, out, buf):
        wid = lax.axis_index("s") * nc + lax.axis_index("c")
        base = wid * rows_per_w
        n_src = BATCH // rows_per_w

        for i in range(rows_per_w // rchunk):
            r = base + i * rchunk

            @pl.when(wid < n_src)
            def _():
                pltpu.sync_copy(kasf.at[pl.ds(r, rchunk)], buf)

            @pl.when((wid >= n_src) & (wid < 2 * n_src))
            def _():
                pltpu.sync_copy(kadf1.at[pl.ds(r - BATCH, rchunk)], buf)

            @pl.when((wid >= 2 * n_src) & (wid < 3 * n_src))
            def _():
                pltpu.sync_copy(kadf2.at[pl.ds(r - 2 * BATCH, rchunk)], buf)

            @pl.when(wid >= 3 * n_src)
            def _():
                pltpu.sync_copy(mem.at[pl.ds(r, rchunk)], buf)

            pltpu.sync_copy(buf, out.at[pl.ds(r, rchunk)])

    return upd(k_all_sf, k_all_df1, k_all_df2, memory)


def kernel(q, k_sf, k_df1, k_df2, k_all_sf, k_all_df1, k_all_df2, memory):
    new_memory = _queue_update_sc(k_all_sf, k_all_df1, k_all_df2, memory)
    _full = lambda c: (0, 0)
    out, l_pos_sf = pl.pallas_call(
        _body,
        grid=(NCB,),
        in_specs=[
            pl.BlockSpec((BATCH, FEAT), _full),
            pl.BlockSpec((CW, FEAT), lambda c: (jnp.minimum(c, NMB - 1), 0)),
            pl.BlockSpec((BATCH, FEAT), _full),
            pl.BlockSpec((BATCH, FEAT), _full),
            pl.BlockSpec((BATCH, FEAT), _full),
        ],
        out_specs=[
            pl.BlockSpec((3 * BATCH, CW), lambda c: (0, c)),
            pl.BlockSpec((BATCH, 1), _full),
        ],
        out_shape=[
            jax.ShapeDtypeStruct((3 * BATCH, QS + 1), jnp.float32),
            jax.ShapeDtypeStruct((BATCH, 1), jnp.float32),
        ],
        scratch_shapes=[pltpu.VMEM((1, FEAT), jnp.float32)],
    )(q, memory, k_sf, k_df1, k_df2)

    return (out, l_pos_sf, new_memory)


# speedup vs baseline: 1.0903x; 1.0903x over previous
import jax
import jax.numpy as jnp
from jax import lax
from jax.experimental import pallas as pl
from jax.experimental.pallas import tpu as pltpu

FEAT = 256
QS = 16384
BATCH = 1024
INV_T = 1.0 / 0.07
CW = 1024
NCB = (QS + 1 + CW - 1) // CW
NMB = QS // CW


def _body(q_ref, mem_ref, ksf_ref, kdf1_ref, kdf2_ref,
          kasf_ref, kadf1_ref, kadf2_ref,
          out_ref, lpos_ref, nm_ref, carry_ref):
    c = pl.program_id(0)
    q = q_ref[...]
    m = mem_ref[...]

    prev = jnp.where(c == 0, jnp.zeros((1, FEAT), jnp.float32), carry_ref[...])
    rhs = jnp.concatenate([prev, m[:CW - 1]], axis=0)
    carry_ref[...] = m[CW - 1:CW]

    tile = jax.lax.dot_general(
        q, rhs, (((1,), (1,)), ((), ())),
        preferred_element_type=jnp.float32,
    ) * INV_T
    out_ref[0:BATCH] = tile
    out_ref[BATCH:2 * BATCH] = tile
    out_ref[2 * BATCH:3 * BATCH] = tile

    @pl.when(c == 0)
    def _():
        psf = jnp.sum(q * ksf_ref[...], axis=1, keepdims=True)
        pd1 = jnp.sum(q * kdf1_ref[...], axis=1, keepdims=True)
        pd2 = jnp.sum(q * kdf2_ref[...], axis=1, keepdims=True)
        lpos_ref[...] = psf
        out_ref[0:BATCH, 0:1] = psf * INV_T
        out_ref[BATCH:2 * BATCH, 0:1] = pd1 * INV_T
        out_ref[2 * BATCH:3 * BATCH, 0:1] = pd2 * INV_T

    @pl.when(c == 0)
    def _():
        nm_ref[...] = kasf_ref[...]

    @pl.when(c == 1)
    def _():
        nm_ref[...] = kadf1_ref[...]

    @pl.when(c == 2)
    def _():
        nm_ref[...] = kadf2_ref[...]

    @pl.when(c >= 3)
    def _():
        nm_ref[...] = m


def kernel(q, k_sf, k_df1, k_df2, k_all_sf, k_all_df1, k_all_df2, memory):
    _full = lambda c: (0, 0)
    out, l_pos_sf, new_memory = pl.pallas_call(
        _body,
        grid=(NCB,),
        in_specs=[
            pl.BlockSpec((BATCH, FEAT), _full),
            pl.BlockSpec((CW, FEAT), lambda c: (jnp.minimum(c, NMB - 1), 0)),
            pl.BlockSpec((BATCH, FEAT), _full),
            pl.BlockSpec((BATCH, FEAT), _full),
            pl.BlockSpec((BATCH, FEAT), _full),
            pl.BlockSpec((BATCH, FEAT), _full),
            pl.BlockSpec((BATCH, FEAT), _full),
            pl.BlockSpec((BATCH, FEAT), _full),
        ],
        out_specs=[
            pl.BlockSpec((3 * BATCH, CW), lambda c: (0, c)),
            pl.BlockSpec((BATCH, 1), _full),
            pl.BlockSpec((CW, FEAT), lambda c: (jnp.minimum(c, NMB - 1), 0)),
        ],
        out_shape=[
            jax.ShapeDtypeStruct((3 * BATCH, QS + 1), jnp.float32),
            jax.ShapeDtypeStruct((BATCH, 1), jnp.float32),
            jax.ShapeDtypeStruct((QS, FEAT), jnp.float32),
        ],
        scratch_shapes=[pltpu.VMEM((1, FEAT), jnp.float32)],
    )(q, memory, k_sf, k_df1, k_df2, k_all_sf, k_all_df1, k_all_df2)

    return (out, l_pos_sf, new_memory)
